# Initial kernel scaffold; baseline (speedup 1.0000x reference)
import jax
import jax.numpy as jnp
from jax.experimental import pallas as pl


def kernel(support_xf, support_y, query_xf, query_y):
    raise NotImplementedError("write your pallas kernel here")


# speedup vs baseline: 4.7462x; 4.7462x over previous
import jax
import jax.numpy as jnp
from jax import lax
from jax.experimental import pallas as pl
from jax.experimental.pallas import tpu as pltpu

N_WAY = 5
K_SHOT = 5
TEMP = 2.0
MS = 500
MSP = 512
M = 100
QB = 5


def _rowify(col):
    r = col.shape[0]
    ones = jnp.ones((r, 1), jnp.float32)
    return lax.dot_general(ones, col, (((1,), (1,)), ((), ())),
                           precision=lax.Precision.HIGHEST,
                           preferred_element_type=jnp.float32)


def _body(sup_ref, qry_ref, lab_ref, out_ref):
    R = QB * M
    sup = sup_ref[0]
    snorm = jnp.sqrt(jnp.sum(sup * sup, axis=0, keepdims=True))
    sn = sup / (snorm + 1e-8)
    qv = qry_ref[0]
    qnorm = jnp.sqrt(jnp.sum(qv * qv, axis=1, keepdims=True))
    qn = qv / (qnorm + 1e-8)

    ci = lax.broadcasted_iota(jnp.int32, (R, MSP), 1)
    colpad = ci >= MS
    cm_cols = []
    am_cols = []
    for n in range(N_WAY):
        s_n = lax.dot_general(qn, sn[:, n * MSP:(n + 1) * MSP],
                              (((1,), (0,)), ((), ())),
                              precision=lax.Precision.HIGHEST,
                              preferred_element_type=jnp.float32)
        s_n = jnp.where(colpad, -2.0, s_n)
        rmax_n = jnp.max(s_n, axis=1, keepdims=True)
        am_n = jnp.min(jnp.where(s_n == rmax_n, ci, MSP),
                       axis=1, keepdims=True)
        cm_cols.append(rmax_n)
        am_cols.append(am_n)

    cm = jnp.concatenate(cm_cols, axis=1)
    t1 = jnp.max(cm, axis=1, keepdims=True)
    i5 = lax.broadcasted_iota(jnp.int32, (R, N_WAY), 1)
    first = jnp.min(jnp.where(cm == t1, i5, N_WAY), axis=1, keepdims=True)
    t2 = jnp.max(jnp.where(i5 == first, -3.0, cm), axis=1, keepdims=True)
    dff = t1 - t2

    kcol = jnp.full((R, 1), jnp.float32(N_WAY * MSP))
    for n in range(N_WAY):
        cand = jnp.where(cm[:, n:n + 1] == t1,
                         (am_cols[n] + n * MSP).astype(jnp.float32),
                         jnp.float32(N_WAY * MSP))
        kcol = jnp.minimum(kcol, cand)

    ri = lax.broadcasted_iota(jnp.int32, (R, R), 0)
    cj = lax.broadcasted_iota(jnp.int32, (R, R), 1)
    same_q = (ri // M) == (cj // M)
    ki_m = jnp.broadcast_to(kcol, (R, R))
    di_m = jnp.broadcast_to(dff, (R, R))
    kj_m = _rowify(kcol)
    dj_m = _rowify(dff)
    contrib = same_q & (kj_m == ki_m) & (
        (dj_m > di_m) | ((dj_m == di_m) & (cj < ri)))
    beaten = jnp.max(jnp.where(contrib, 1.0, 0.0), axis=1, keepdims=True)
    pos0 = (lax.broadcasted_iota(jnp.int32, (R, 1), 0) % M) == 0
    maskp = (beaten < 0.5) & ((dff > 0.0) | pos0)

    z = cm * jnp.where(maskp, 1.0, 0.0)
    gi = lax.broadcasted_iota(jnp.int32, (QB, R), 0)
    gj = lax.broadcasted_iota(jnp.int32, (QB, R), 1)
    sel = jnp.where(gi == gj // M, 1.0, 0.0)
    qvals = lax.dot_general(sel, z, (((1,), (0,)), ((), ())),
                            precision=lax.Precision.HIGHEST,
                            preferred_element_type=jnp.float32)

    lab = lab_ref[0]
    logits = qvals * (1.0 / TEMP)
    mx = jnp.max(logits, axis=1, keepdims=True)
    lse = jnp.log(jnp.sum(jnp.exp(logits - mx), axis=1, keepdims=True))
    i5b = lax.broadcasted_iota(jnp.int32, (QB, N_WAY), 1)
    picked = jnp.sum(jnp.where(i5b == lab, logits - mx, 0.0),
                     axis=1, keepdims=True)
    part = jnp.sum(lse - picked, keepdims=True)

    @pl.when((pl.program_id(0) == 0) & (pl.program_id(1) == 0))
    def _init():
        out_ref[...] = jnp.zeros((1, 1), jnp.float32)

    out_ref[...] += part


def kernel(support_xf, support_y, query_xf, query_y):
    b, s, c, h, w = support_xf.shape
    q = query_xf.shape[1]
    m = h * w
    nqb = q // QB

    sup = support_xf.reshape(b, N_WAY, K_SHOT, c, m)
    sup = jnp.transpose(sup, (0, 3, 1, 2, 4)).reshape(b, c, N_WAY, K_SHOT * m)
    sup = jnp.pad(sup, ((0, 0), (0, 0), (0, 0), (0, MSP - K_SHOT * m)))
    sup = sup.reshape(b, c, N_WAY * MSP)

    qry = jnp.transpose(query_xf.reshape(b, q, c, m), (0, 1, 3, 2))
    qry = qry.reshape(b * nqb, QB * m, c)

    labs = query_y.reshape(b * nqb, QB, 1)

    total = pl.pallas_call(
        _body,
        grid=(b, nqb),
        in_specs=[
            pl.BlockSpec((1, c, N_WAY * MSP), lambda bi, qi: (bi, 0, 0)),
            pl.BlockSpec((1, QB * m, c), lambda bi, qi: (bi * (q // QB) + qi, 0, 0)),
            pl.BlockSpec((1, QB, 1), lambda bi, qi: (bi * (q // QB) + qi, 0, 0)),
        ],
        out_specs=pl.BlockSpec((1, 1), lambda bi, qi: (0, 0)),
        out_shape=jax.ShapeDtypeStruct((1, 1), jnp.float32),
        compiler_params=pltpu.CompilerParams(
            dimension_semantics=("arbitrary", "arbitrary")),
    )(sup, qry, labs)
    return total[0, 0] / (b * q)


# speedup vs baseline: 5.8156x; 1.2253x over previous
import functools

import jax
import jax.numpy as jnp
from jax import lax
from jax.experimental import pallas as pl
from jax.experimental.pallas import tpu as pltpu

N_WAY = 5
K_SHOT = 5
TEMP = 2.0
MS = 500
MSP = 512
M = 100
QB = 5
R = QB * M


def _rowify(col):
    r = col.shape[0]
    ones = jnp.ones((r, 1), jnp.float32)
    return lax.dot_general(ones, col, (((1,), (1,)), ((), ())),
                           precision=lax.Precision.HIGHEST,
                           preferred_element_type=jnp.float32)


def _body(nb, sup_ref, qry_ref, lab_ref, out_ref, buf_ref):
    t = pl.program_id(0)
    wslot = t % 2
    rslot = (t + 1) % 2

    sup = sup_ref[0]
    snorm = jnp.sqrt(jnp.sum(sup * sup, axis=0, keepdims=True))
    sn = sup / (snorm + 1e-8)
    qv = qry_ref[0]
    qnorm = jnp.sqrt(jnp.sum(qv * qv, axis=1, keepdims=True))
    qn = qv / (qnorm + 1e-8)
    colpad = lax.broadcasted_iota(jnp.int32, (R, MSP), 1) >= MS
    cif = lax.broadcasted_iota(jnp.int32, (R, MSP), 1).astype(jnp.float32)

    cm_cols = []
    am_cols = []
    for n in range(N_WAY):
        s_old = buf_ref[rslot, :, n * MSP:(n + 1) * MSP]
        rmax_n = jnp.max(s_old, axis=1, keepdims=True)
        am_n = jnp.min(jnp.where(s_old == rmax_n, cif, float(MSP)),
                       axis=1, keepdims=True)
        cm_cols.append(rmax_n)
        am_cols.append(am_n)
        s_new = lax.dot_general(qn, sn[:, n * MSP:(n + 1) * MSP],
                                (((1,), (0,)), ((), ())),
                                precision=lax.Precision.HIGHEST,
                                preferred_element_type=jnp.float32)
        buf_ref[wslot, :, n * MSP:(n + 1) * MSP] = \
            jnp.where(colpad, -2.0, s_new)

    cm = jnp.concatenate(cm_cols, axis=1)
    t1 = jnp.max(cm, axis=1, keepdims=True)
    i5 = lax.broadcasted_iota(jnp.int32, (R, N_WAY), 1).astype(jnp.float32)
    first = jnp.min(jnp.where(cm == t1, i5, float(N_WAY)),
                    axis=1, keepdims=True)
    t2 = jnp.max(jnp.where(i5 == first, -3.0, cm), axis=1, keepdims=True)
    dff = t1 - t2

    kcol = jnp.full((R, 1), jnp.float32(N_WAY * MSP))
    for n in range(N_WAY):
        cand = jnp.where(cm[:, n:n + 1] == t1, am_cols[n] + float(n * MSP),
                         jnp.float32(N_WAY * MSP))
        kcol = jnp.minimum(kcol, cand)

    ri = lax.broadcasted_iota(jnp.int32, (M, M), 0)
    cj = lax.broadcasted_iota(jnp.int32, (M, M), 1)
    ltm = (cj < ri)
    mask_cols = []
    for g in range(QB):
        kg = lax.slice(kcol, (g * M, 0), ((g + 1) * M, 1))
        dg = lax.slice(dff, (g * M, 0), ((g + 1) * M, 1))
        kj_m = _rowify(kg)
        dj_m = _rowify(dg)
        contrib = (kj_m == kg) & ((dj_m > dg) | ((dj_m == dg) & ltm))
        beaten = jnp.max(jnp.where(contrib, 1.0, 0.0),
                         axis=1, keepdims=True)
        pos0 = lax.broadcasted_iota(jnp.int32, (M, 1), 0) == 0
        keep = (beaten < 0.5) & ((dg > 0.0) | pos0)
        mask_cols.append(jnp.where(keep, 1.0, 0.0))
    maskp = jnp.concatenate(mask_cols, axis=0)

    z = cm * maskp
    gi = lax.broadcasted_iota(jnp.int32, (QB, R), 0)
    gj = lax.broadcasted_iota(jnp.int32, (QB, R), 1)
    sel = jnp.where(gi == gj // M, 1.0, 0.0)
    qvals = lax.dot_general(sel, z, (((1,), (0,)), ((), ())),
                            precision=lax.Precision.HIGHEST,
                            preferred_element_type=jnp.float32)

    lab = lab_ref[0]
    logits = qvals * (1.0 / TEMP)
    mx = jnp.max(logits, axis=1, keepdims=True)
    lse = jnp.log(jnp.sum(jnp.exp(logits - mx), axis=1, keepdims=True))
    i5b = lax.broadcasted_iota(jnp.int32, (QB, N_WAY), 1)
    picked = jnp.sum(jnp.where(i5b == lab, logits - mx, 0.0),
                     axis=1, keepdims=True)
    part = jnp.sum(lse - picked, keepdims=True)

    prev = jnp.where(t == 0, jnp.zeros((1, 1), jnp.float32), out_ref[...])
    out_ref[...] = prev + jnp.where(t > 0, part, 0.0)


def kernel(support_xf, support_y, query_xf, query_y):
    b, s, c, h, w = support_xf.shape
    q = query_xf.shape[1]
    m = h * w
    nqb = q // QB
    nb = b * nqb

    sup = support_xf.reshape(b, N_WAY, K_SHOT, c, m)
    sup = jnp.transpose(sup, (0, 3, 1, 2, 4)).reshape(b, c, N_WAY, K_SHOT * m)
    sup = jnp.pad(sup, ((0, 0), (0, 0), (0, 0), (0, MSP - K_SHOT * m)))
    sup = sup.reshape(b, c, N_WAY * MSP)

    qry = jnp.transpose(query_xf.reshape(b, q, c, m), (0, 1, 3, 2))
    qry = qry.reshape(nb, R, c)

    labs = query_y.reshape(nb, QB, 1)

    total = pl.pallas_call(
        functools.partial(_body, nb),
        grid=(nb + 1,),
        in_specs=[
            pl.BlockSpec((1, c, N_WAY * MSP),
                         lambda t: (jnp.minimum(t, nb - 1) // nqb, 0, 0)),
            pl.BlockSpec((1, R, c), lambda t: (jnp.minimum(t, nb - 1), 0, 0)),
            pl.BlockSpec((1, QB, 1), lambda t: (jnp.maximum(t - 1, 0), 0, 0)),
        ],
        out_specs=pl.BlockSpec((1, 1), lambda t: (0, 0)),
        out_shape=jax.ShapeDtypeStruct((1, 1), jnp.float32),
        scratch_shapes=[pltpu.VMEM((2, R, N_WAY * MSP), jnp.float32)],
        compiler_params=pltpu.CompilerParams(
            dimension_semantics=("arbitrary",)),
    )(sup, qry, labs)
    return total[0, 0] / (b * q)
